# Initial kernel scaffold; baseline (speedup 1.0000x reference)
import jax
import jax.numpy as jnp
from jax.experimental import pallas as pl


def kernel(input_nodes, in_degree, out_degree, atom_table, in_table, out_table, graph_token):
    raise NotImplementedError("write your pallas kernel here")


# speedup vs baseline: 3.2519x; 3.2519x over previous
import functools

import jax
import jax.numpy as jnp
from jax import lax
from jax.experimental import pallas as pl
from jax.experimental.pallas import tpu as pltpu
from jax.experimental.pallas import tpu_sc as plsc

H = 768
NC = 2
NS = 16
NW = NC * NS
K = 8
IDX_PER_CHUNK = K * 11
LANES = 16


def _sc_sum_kernel(total_rows: int, cpw: int):
    mesh = plsc.VectorSubcoreMesh(core_axis_name="c", subcore_axis_name="s")

    @functools.partial(
        pl.kernel,
        mesh=mesh,
        out_type=jax.ShapeDtypeStruct((total_rows, H), jnp.float32),
        scratch_types=[
            pltpu.VMEM((cpw, IDX_PER_CHUNK), jnp.int32),
            pltpu.VMEM((IDX_PER_CHUNK, H), jnp.float32),
            pltpu.VMEM((K, H), jnp.float32),
            pltpu.SemaphoreType.DMA,
            pltpu.SemaphoreType.DMA,
        ],
    )
    def k(table_hbm, gidx_hbm, out_hbm, idx_v, rows_v, out_v, sem_g, sem_o):
        wid = lax.axis_index("s") * NC + lax.axis_index("c")
        pltpu.sync_copy(gidx_hbm.at[wid], idx_v)

        def chunk_body(t, carry):
            pltpu.async_copy(table_hbm.at[idx_v.at[t]], rows_v, sem_g).wait()

            def col_body(c, cc):
                for kk in range(K):
                    acc = rows_v[kk * 11, pl.ds(c * LANES, LANES)]
                    for j in range(1, 11):
                        acc = acc + rows_v[kk * 11 + j, pl.ds(c * LANES, LANES)]
                    out_v[kk, pl.ds(c * LANES, LANES)] = acc
                return cc

            lax.fori_loop(0, H // LANES, col_body, 0)
            pltpu.async_copy(
                out_v, out_hbm.at[pl.ds(wid * cpw * K + t * K, K)], sem_o
            ).wait()
            return carry

        lax.fori_loop(0, cpw, chunk_body, 0)

    return k


def kernel(input_nodes, in_degree, out_degree, atom_table, in_table, out_table, graph_token):
    B, N, F = input_nodes.shape
    num_atom = atom_table.shape[0]
    num_in = in_table.shape[0]
    total_rows = B * (N + 1)
    cpw = total_rows // (NW * K)

    table = jnp.concatenate(
        [atom_table, in_table, out_table, graph_token], axis=0
    ).astype(jnp.float32)
    tok_row = num_atom + num_in + out_table.shape[0]

    atoms = input_nodes.astype(jnp.int32)
    node_idx = jnp.concatenate(
        [
            atoms,
            (in_degree.astype(jnp.int32) + num_atom)[..., None],
            (out_degree.astype(jnp.int32) + num_atom + num_in)[..., None],
        ],
        axis=-1,
    )
    tok_idx = jnp.zeros((B, 1, F + 2), jnp.int32).at[:, :, 0].set(tok_row)
    gidx = jnp.concatenate([tok_idx, node_idx], axis=1).reshape(
        NW, cpw, IDX_PER_CHUNK
    )

    out = _sc_sum_kernel(total_rows, cpw)(table, gidx)
    return out.reshape(B, N + 1, H)
